# Initial kernel scaffold; baseline (speedup 1.0000x reference)
import jax
import jax.numpy as jnp
from jax.experimental import pallas as pl


def kernel(x, edge_index, W1, b1, W2, b2):
    raise NotImplementedError("write your pallas kernel here")


# speedup vs baseline: 46.4458x; 46.4458x over previous
import functools

import jax
import jax.numpy as jnp
from jax import lax
from jax.experimental import pallas as pl
from jax.experimental.pallas import tpu as pltpu
from jax.experimental.pallas import tpu_sc as plsc

N_NODES = 100000
N_EDGES = 1600000

NCORES = 2
NSUB = 16
NWORK = NCORES * NSUB

BLK = 1024
NPAD = 100352
STRIPE = NPAD // NSUB

LANE = 128
INNER = 8
OUTER = 50
ROWS_PER_WORKER = INNER * OUTER
EPAD = NWORK * ROWS_PER_WORKER * LANE
EROWS = EPAD // LANE


def _sc_mesh():
    return plsc.VectorSubcoreMesh(core_axis_name="c", subcore_axis_name="s")


def _sc_agg(row2d, col2d, ys, zeros1, ones_h, with_deg):
    nf = len(ys)
    nout = nf + (1 if with_deg else 0)
    out_t = [jax.ShapeDtypeStruct((NCORES * NPAD,), jnp.float32)] * nout
    scratch = (
        [pltpu.VMEM((INNER, LANE), jnp.int32)] * 2
        + [pltpu.VMEM((LANE,), jnp.float32) for _ in range(nf)]
        + [pltpu.VMEM((LANE,), jnp.float32)]
        + [pltpu.VMEM((STRIPE,), jnp.float32)]
        + [pltpu.VMEM_SHARED((NPAD,), jnp.float32) for _ in range(nf + nout)]
    )

    @functools.partial(
        pl.kernel,
        mesh=_sc_mesh(),
        out_type=out_t,
        scratch_types=scratch,
    )
    def k(row_hbm, col_hbm, *rest):
        y_hbm = rest[:nf]
        z_hbm = rest[nf]
        ones_hbm = rest[nf + 1]
        outs = rest[nf + 2:nf + 2 + nout]
        rowv, colv = rest[nf + 2 + nout:nf + 4 + nout]
        gbufs = rest[nf + 4 + nout:nf + 4 + nout + nf]
        onesv = rest[nf + 4 + nout + nf]
        stripev = rest[nf + 5 + nout + nf]
        ysp = rest[nf + 6 + nout + nf:nf + 6 + nout + 2 * nf]
        accs = rest[nf + 6 + nout + 2 * nf:]

        cid = lax.axis_index("c")
        sid = lax.axis_index("s")
        t0 = sid * STRIPE
        sl = pl.ds(t0, STRIPE)
        pltpu.sync_copy(z_hbm.at[sl], stripev)
        for a in accs:
            pltpu.sync_copy(stripev, a.at[sl])
        for yh, yp in zip(y_hbm, ysp):
            pltpu.sync_copy(yh.at[sl], stripev)
            pltpu.sync_copy(stripev, yp.at[sl])
        pltpu.sync_copy(ones_hbm, onesv)
        plsc.subcore_barrier()

        base = (cid * NSUB + sid) * ROWS_PER_WORKER

        def body(b, carry):
            pltpu.sync_copy(row_hbm.at[pl.ds(base + b * INNER, INNER)], rowv)
            pltpu.sync_copy(col_hbm.at[pl.ds(base + b * INNER, INNER)], colv)
            for j in range(INNER):
                for f in range(nf):
                    pltpu.sync_copy(ysp[f].at[rowv.at[j]], gbufs[f])
                    pltpu.sync_copy(gbufs[f], accs[f].at[colv.at[j]],
                                    add=True)
                if with_deg:
                    pltpu.sync_copy(onesv, accs[nf].at[colv.at[j]], add=True)
            return carry

        lax.fori_loop(0, OUTER, body, 0)
        plsc.subcore_barrier()
        for a, o in zip(accs, outs):
            pltpu.sync_copy(a.at[sl], stripev)
            pltpu.sync_copy(stripev, o.at[pl.ds(cid * NPAD + t0, STRIPE)])

    outs = k(row2d, col2d, *ys, zeros1, ones_h)
    return outs if isinstance(outs, (list, tuple)) else [outs]


def _tc1(degp, xpad):
    def body(degp_ref, x_ref, dis_ref, y0_ref, y1_ref):
        deg = degp_ref[0, :] + degp_ref[1, :]
        dis = jnp.where(deg > 0, lax.rsqrt(deg), 0.0)
        dis_ref[...] = dis
        y0_ref[...] = dis * x_ref[:, 0]
        y1_ref[...] = dis * x_ref[:, 1]

    return pl.pallas_call(
        body,
        grid=(NPAD // BLK,),
        in_specs=[
            pl.BlockSpec((NCORES, BLK), lambda i: (0, i)),
            pl.BlockSpec((BLK, 2), lambda i: (i, 0)),
        ],
        out_specs=[
            pl.BlockSpec((BLK,), lambda i: (i,)),
            pl.BlockSpec((BLK,), lambda i: (i,)),
            pl.BlockSpec((BLK,), lambda i: (i,)),
        ],
        out_shape=[
            jax.ShapeDtypeStruct((NPAD,), jnp.float32),
            jax.ShapeDtypeStruct((NPAD,), jnp.float32),
            jax.ShapeDtypeStruct((NPAD,), jnp.float32),
        ],
    )(degp, xpad)


def _tc2(s10, s11, dis, W1, b1, W2):
    def body(s10_ref, s11_ref, dis_ref, w1_ref, b1_ref, w2_ref, y2_ref):
        a0 = s10_ref[0, :] + s10_ref[1, :]
        a1 = s11_ref[0, :] + s11_ref[1, :]
        dis = dis_ref[...]
        a0 = dis * a0
        a1 = dis * a1
        h = (a0[:, None] * w1_ref[0:1, :]
             + a1[:, None] * w1_ref[1:2, :]
             + b1_ref[...][None, :])
        h = jnp.maximum(h, 0.0)
        v = jnp.sum(h * w2_ref[:, 0][None, :], axis=1)
        y2_ref[...] = dis * v

    return pl.pallas_call(
        body,
        grid=(NPAD // BLK,),
        in_specs=[
            pl.BlockSpec((NCORES, BLK), lambda i: (0, i)),
            pl.BlockSpec((NCORES, BLK), lambda i: (0, i)),
            pl.BlockSpec((BLK,), lambda i: (i,)),
            pl.BlockSpec((2, 64), lambda i: (0, 0)),
            pl.BlockSpec((64,), lambda i: (0,)),
            pl.BlockSpec((64, 1), lambda i: (0, 0)),
        ],
        out_specs=pl.BlockSpec((BLK,), lambda i: (i,)),
        out_shape=jax.ShapeDtypeStruct((NPAD,), jnp.float32),
    )(s10, s11, dis, W1, b1, W2)


def _tc3(s2p, dis, b2):
    def body(s2p_ref, dis_ref, b2_ref, out_ref):
        s2 = s2p_ref[0, :] + s2p_ref[1, :]
        out = dis_ref[...] * s2 + b2_ref[0]
        out_ref[...] = jnp.maximum(out, 0.0)[:, None]

    return pl.pallas_call(
        body,
        grid=(NPAD // BLK,),
        in_specs=[
            pl.BlockSpec((NCORES, BLK), lambda i: (0, i)),
            pl.BlockSpec((BLK,), lambda i: (i,)),
            pl.BlockSpec((1,), lambda i: (0,)),
        ],
        out_specs=pl.BlockSpec((BLK, 1), lambda i: (i, 0)),
        out_shape=jax.ShapeDtypeStruct((NPAD, 1), jnp.float32),
    )(s2p, dis, b2)


def kernel(x, edge_index, W1, b1, W2, b2):
    pad = jnp.full((EPAD - N_EDGES,), N_NODES, jnp.int32)
    row2d = jnp.concatenate([edge_index[0], pad]).reshape(EROWS, LANE)
    col2d = jnp.concatenate([edge_index[1], pad]).reshape(EROWS, LANE)
    xpad = jnp.zeros((NPAD, 2), jnp.float32).at[:N_NODES].set(x)
    z1 = jnp.zeros((NPAD,), jnp.float32)
    ones_h = jnp.ones((LANE,), jnp.float32)

    (degp,) = _sc_agg(row2d, col2d, [], z1, ones_h, True)
    dis, y0, y1 = _tc1(degp.reshape(NCORES, NPAD), xpad)
    s10, s11 = _sc_agg(row2d, col2d, [y0, y1], z1, ones_h, False)
    y2 = _tc2(s10.reshape(NCORES, NPAD), s11.reshape(NCORES, NPAD),
              dis, W1, b1, W2)
    (s2p,) = _sc_agg(row2d, col2d, [y2], z1, ones_h, False)
    out = _tc3(s2p.reshape(NCORES, NPAD), dis, b2)
    return out[:N_NODES]


# speedup vs baseline: 53.3858x; 1.1494x over previous
import functools

import jax
import jax.numpy as jnp
from jax import lax
from jax.experimental import pallas as pl
from jax.experimental.pallas import tpu as pltpu
from jax.experimental.pallas import tpu_sc as plsc

N_NODES = 100000
N_EDGES = 1600000

NCORES = 2
NSUB = 16
NWORK = NCORES * NSUB

BLK = 1024
NPAD = 100352
STRIPE = NPAD // NSUB

CH = 1024
OUTER = 50
EDGES_PER_WORKER = CH * OUTER
EPAD = NWORK * EDGES_PER_WORKER


def _sc_mesh():
    return plsc.VectorSubcoreMesh(core_axis_name="c", subcore_axis_name="s")


def _sc_agg(row1d, col1d, ys, zeros1, ones_h, with_deg):
    nf = len(ys)
    nout = nf + (1 if with_deg else 0)
    out_t = [jax.ShapeDtypeStruct((NCORES * NPAD,), jnp.float32)] * nout
    scratch = (
        [pltpu.VMEM((CH,), jnp.int32)] * 2
        + [pltpu.VMEM((CH,), jnp.float32) for _ in range(nf)]
        + [pltpu.VMEM((CH,), jnp.float32)]
        + [pltpu.VMEM((STRIPE,), jnp.float32)]
        + [pltpu.VMEM_SHARED((NPAD,), jnp.float32) for _ in range(nf + nout)]
    )

    @functools.partial(
        pl.kernel,
        mesh=_sc_mesh(),
        out_type=out_t,
        scratch_types=scratch,
    )
    def k(row_hbm, col_hbm, *rest):
        y_hbm = rest[:nf]
        z_hbm = rest[nf]
        ones_hbm = rest[nf + 1]
        outs = rest[nf + 2:nf + 2 + nout]
        rowv, colv = rest[nf + 2 + nout:nf + 4 + nout]
        gbufs = rest[nf + 4 + nout:nf + 4 + nout + nf]
        onesv = rest[nf + 4 + nout + nf]
        stripev = rest[nf + 5 + nout + nf]
        ysp = rest[nf + 6 + nout + nf:nf + 6 + nout + 2 * nf]
        accs = rest[nf + 6 + nout + 2 * nf:]

        cid = lax.axis_index("c")
        sid = lax.axis_index("s")
        t0 = sid * STRIPE
        sl = pl.ds(t0, STRIPE)
        pltpu.sync_copy(z_hbm.at[sl], stripev)
        for a in accs:
            pltpu.sync_copy(stripev, a.at[sl])
        for yh, yp in zip(y_hbm, ysp):
            pltpu.sync_copy(yh.at[sl], stripev)
            pltpu.sync_copy(stripev, yp.at[sl])
        if with_deg:
            pltpu.sync_copy(ones_hbm, onesv)
        plsc.subcore_barrier()

        base = (cid * NSUB + sid) * EDGES_PER_WORKER

        def body(b, carry):
            off = pl.ds(base + b * CH, CH)
            if nf:
                pltpu.sync_copy(row_hbm.at[off], rowv)
            pltpu.sync_copy(col_hbm.at[off], colv)
            for f in range(nf):
                pltpu.sync_copy(ysp[f].at[rowv], gbufs[f])
                pltpu.sync_copy(gbufs[f], accs[f].at[colv], add=True)
            if with_deg:
                pltpu.sync_copy(onesv, accs[nf].at[colv], add=True)
            return carry

        lax.fori_loop(0, OUTER, body, 0)
        plsc.subcore_barrier()
        for a, o in zip(accs, outs):
            pltpu.sync_copy(a.at[sl], stripev)
            pltpu.sync_copy(stripev, o.at[pl.ds(cid * NPAD + t0, STRIPE)])

    outs = k(row1d, col1d, *ys, zeros1, ones_h)
    return outs if isinstance(outs, (list, tuple)) else [outs]


def _tc1(degp, xpad):
    def body(degp_ref, x_ref, dis_ref, y0_ref, y1_ref):
        deg = degp_ref[0, :] + degp_ref[1, :]
        dis = jnp.where(deg > 0, lax.rsqrt(deg), 0.0)
        dis_ref[...] = dis
        y0_ref[...] = dis * x_ref[:, 0]
        y1_ref[...] = dis * x_ref[:, 1]

    return pl.pallas_call(
        body,
        grid=(NPAD // BLK,),
        in_specs=[
            pl.BlockSpec((NCORES, BLK), lambda i: (0, i)),
            pl.BlockSpec((BLK, 2), lambda i: (i, 0)),
        ],
        out_specs=[
            pl.BlockSpec((BLK,), lambda i: (i,)),
            pl.BlockSpec((BLK,), lambda i: (i,)),
            pl.BlockSpec((BLK,), lambda i: (i,)),
        ],
        out_shape=[
            jax.ShapeDtypeStruct((NPAD,), jnp.float32),
            jax.ShapeDtypeStruct((NPAD,), jnp.float32),
            jax.ShapeDtypeStruct((NPAD,), jnp.float32),
        ],
    )(degp, xpad)


def _tc2(s10, s11, dis, W1, b1, W2):
    def body(s10_ref, s11_ref, dis_ref, w1_ref, b1_ref, w2_ref, y2_ref):
        a0 = s10_ref[0, :] + s10_ref[1, :]
        a1 = s11_ref[0, :] + s11_ref[1, :]
        dis = dis_ref[...]
        a0 = dis * a0
        a1 = dis * a1
        h = (a0[:, None] * w1_ref[0:1, :]
             + a1[:, None] * w1_ref[1:2, :]
             + b1_ref[...][None, :])
        h = jnp.maximum(h, 0.0)
        v = jnp.sum(h * w2_ref[:, 0][None, :], axis=1)
        y2_ref[...] = dis * v

    return pl.pallas_call(
        body,
        grid=(NPAD // BLK,),
        in_specs=[
            pl.BlockSpec((NCORES, BLK), lambda i: (0, i)),
            pl.BlockSpec((NCORES, BLK), lambda i: (0, i)),
            pl.BlockSpec((BLK,), lambda i: (i,)),
            pl.BlockSpec((2, 64), lambda i: (0, 0)),
            pl.BlockSpec((64,), lambda i: (0,)),
            pl.BlockSpec((64, 1), lambda i: (0, 0)),
        ],
        out_specs=pl.BlockSpec((BLK,), lambda i: (i,)),
        out_shape=jax.ShapeDtypeStruct((NPAD,), jnp.float32),
    )(s10, s11, dis, W1, b1, W2)


def _tc3(s2p, dis, b2):
    def body(s2p_ref, dis_ref, b2_ref, out_ref):
        s2 = s2p_ref[0, :] + s2p_ref[1, :]
        out = dis_ref[...] * s2 + b2_ref[0]
        out_ref[...] = jnp.maximum(out, 0.0)[:, None]

    return pl.pallas_call(
        body,
        grid=(NPAD // BLK,),
        in_specs=[
            pl.BlockSpec((NCORES, BLK), lambda i: (0, i)),
            pl.BlockSpec((BLK,), lambda i: (i,)),
            pl.BlockSpec((1,), lambda i: (0,)),
        ],
        out_specs=pl.BlockSpec((BLK, 1), lambda i: (i, 0)),
        out_shape=jax.ShapeDtypeStruct((NPAD, 1), jnp.float32),
    )(s2p, dis, b2)


def kernel(x, edge_index, W1, b1, W2, b2):
    pad = jnp.full((EPAD - N_EDGES,), N_NODES, jnp.int32)
    row1d = jnp.concatenate([edge_index[0], pad])
    col1d = jnp.concatenate([edge_index[1], pad])
    xpad = jnp.zeros((NPAD, 2), jnp.float32).at[:N_NODES].set(x)
    z1 = jnp.zeros((NPAD,), jnp.float32)
    ones_h = jnp.ones((CH,), jnp.float32)

    (degp,) = _sc_agg(row1d, col1d, [], z1, ones_h, True)
    dis, y0, y1 = _tc1(degp.reshape(NCORES, NPAD), xpad)
    s10, s11 = _sc_agg(row1d, col1d, [y0, y1], z1, ones_h, False)
    y2 = _tc2(s10.reshape(NCORES, NPAD), s11.reshape(NCORES, NPAD),
              dis, W1, b1, W2)
    (s2p,) = _sc_agg(row1d, col1d, [y2], z1, ones_h, False)
    out = _tc3(s2p.reshape(NCORES, NPAD), dis, b2)
    return out[:N_NODES]


# speedup vs baseline: 136.4214x; 2.5554x over previous
import functools

import jax
import jax.numpy as jnp
from jax import lax
from jax.experimental import pallas as pl
from jax.experimental.pallas import tpu as pltpu
from jax.experimental.pallas import tpu_sc as plsc

N_NODES = 100000
N_EDGES = 1600000

NCORES = 2
NSUB = 16
NWORK = NCORES * NSUB

NPAD = 100352
R = NPAD // 128
STRIPE = NPAD // NSUB

CH = 2000
OUTER = 25
EDGES_PER_WORKER = CH * OUTER
assert NWORK * EDGES_PER_WORKER == N_EDGES


def _sc_mesh():
    return plsc.VectorSubcoreMesh(core_axis_name="c", subcore_axis_name="s")


def _sc_agg(row1d, col1d, ys, zeros1, ones_h, with_deg):
    nf = len(ys)
    nout = nf + (1 if with_deg else 0)
    out_t = [jax.ShapeDtypeStruct((NCORES * NPAD,), jnp.float32)] * nout
    scratch = (
        [pltpu.VMEM((CH,), jnp.int32)] * 2
        + [pltpu.VMEM((CH,), jnp.float32) for _ in range(nf)]
        + [pltpu.VMEM((CH,), jnp.float32)]
        + [pltpu.VMEM((STRIPE,), jnp.float32)]
        + [pltpu.VMEM_SHARED((NPAD,), jnp.float32) for _ in range(nf + nout)]
    )

    @functools.partial(
        pl.kernel,
        mesh=_sc_mesh(),
        out_type=out_t,
        scratch_types=scratch,
    )
    def k(row_hbm, col_hbm, *rest):
        y_hbm = rest[:nf]
        z_hbm = rest[nf]
        ones_hbm = rest[nf + 1]
        outs = rest[nf + 2:nf + 2 + nout]
        rowv, colv = rest[nf + 2 + nout:nf + 4 + nout]
        gbufs = rest[nf + 4 + nout:nf + 4 + nout + nf]
        onesv = rest[nf + 4 + nout + nf]
        stripev = rest[nf + 5 + nout + nf]
        ysp = rest[nf + 6 + nout + nf:nf + 6 + nout + 2 * nf]
        accs = rest[nf + 6 + nout + 2 * nf:]

        cid = lax.axis_index("c")
        sid = lax.axis_index("s")
        t0 = sid * STRIPE
        sl = pl.ds(t0, STRIPE)
        pltpu.sync_copy(z_hbm.at[sl], stripev)
        for a in accs:
            pltpu.sync_copy(stripev, a.at[sl])
        for yh, yp in zip(y_hbm, ysp):
            pltpu.sync_copy(yh.at[sl], stripev)
            pltpu.sync_copy(stripev, yp.at[sl])
        if with_deg:
            pltpu.sync_copy(ones_hbm, onesv)
        plsc.subcore_barrier()

        wid = cid * NSUB + sid

        def body(b, carry):
            off = pl.ds((b * NWORK + wid) * CH, CH)
            if nf:
                pltpu.sync_copy(row_hbm.at[off], rowv)
            pltpu.sync_copy(col_hbm.at[off], colv)
            for f in range(nf):
                pltpu.sync_copy(ysp[f].at[rowv], gbufs[f])
                pltpu.sync_copy(gbufs[f], accs[f].at[colv], add=True)
            if with_deg:
                pltpu.sync_copy(onesv, accs[nf].at[colv], add=True)
            return carry

        lax.fori_loop(0, OUTER, body, 0)
        plsc.subcore_barrier()
        for a, o in zip(accs, outs):
            pltpu.sync_copy(a.at[sl], stripev)
            pltpu.sync_copy(stripev, o.at[pl.ds(cid * NPAD + t0, STRIPE)])

    outs = k(row1d, col1d, *ys, zeros1, ones_h)
    return outs if isinstance(outs, (list, tuple)) else [outs]


def _tc1(degp, x0, x1):
    def body(degp_ref, x0_ref, x1_ref, dis_ref, y0_ref, y1_ref):
        deg = degp_ref[0] + degp_ref[1]
        dis = jnp.where(deg > 0, lax.rsqrt(deg), 0.0)
        dis_ref[...] = dis
        y0_ref[...] = dis * x0_ref[...]
        y1_ref[...] = dis * x1_ref[...]

    s = jax.ShapeDtypeStruct((R, 128), jnp.float32)
    return pl.pallas_call(body, out_shape=[s, s, s])(degp, x0, x1)


def _tc2(s10, s11, dis, W1, b1, W2):
    def body(s10_ref, s11_ref, dis_ref, w1_ref, b1_ref, w2_ref, y2_ref):
        dis = dis_ref[...]
        a0 = (s10_ref[0] + s10_ref[1]) * dis
        a1 = (s11_ref[0] + s11_ref[1]) * dis
        w1 = w1_ref[...]
        b1 = b1_ref[...]
        w2 = w2_ref[...]
        v = jnp.zeros_like(a0)
        for k in range(64):
            h = jnp.maximum(a0 * w1[0, k] + a1 * w1[1, k] + b1[k], 0.0)
            v = v + h * w2[k, 0]
        y2_ref[...] = dis * v

    RB = R // 7

    return pl.pallas_call(
        body,
        grid=(7,),
        in_specs=[
            pl.BlockSpec((NCORES, RB, 128), lambda i: (0, i, 0)),
            pl.BlockSpec((NCORES, RB, 128), lambda i: (0, i, 0)),
            pl.BlockSpec((RB, 128), lambda i: (i, 0)),
            pl.BlockSpec((2, 64), lambda i: (0, 0)),
            pl.BlockSpec((64,), lambda i: (0,)),
            pl.BlockSpec((64, 1), lambda i: (0, 0)),
        ],
        out_specs=pl.BlockSpec((RB, 128), lambda i: (i, 0)),
        out_shape=jax.ShapeDtypeStruct((R, 128), jnp.float32),
    )(s10, s11, dis, W1, b1, W2)


def _tc3(s2p, dis, b2):
    def body(s2p_ref, dis_ref, b2_ref, out_ref):
        s2 = s2p_ref[0] + s2p_ref[1]
        out_ref[...] = jnp.maximum(dis_ref[...] * s2 + b2_ref[0], 0.0)

    return pl.pallas_call(
        body, out_shape=jax.ShapeDtypeStruct((R, 128), jnp.float32),
    )(s2p, dis, b2)


def kernel(x, edge_index, W1, b1, W2, b2):
    row = edge_index[0]
    col = edge_index[1]
    xpad = jnp.zeros((NPAD, 2), jnp.float32).at[:N_NODES].set(x)
    x0 = xpad[:, 0].reshape(R, 128)
    x1 = xpad[:, 1].reshape(R, 128)
    z1 = jnp.zeros((NPAD,), jnp.float32)
    ones_h = jnp.ones((CH,), jnp.float32)

    (degp,) = _sc_agg(row, col, [], z1, ones_h, True)
    dis, y0, y1 = _tc1(degp.reshape(NCORES, R, 128), x0, x1)
    s10, s11 = _sc_agg(row, col, [y0.reshape(NPAD), y1.reshape(NPAD)],
                       z1, ones_h, False)
    y2 = _tc2(s10.reshape(NCORES, R, 128), s11.reshape(NCORES, R, 128),
              dis, W1, b1, W2)
    (s2p,) = _sc_agg(row, col, [y2.reshape(NPAD)], z1, ones_h, False)
    out = _tc3(s2p.reshape(NCORES, R, 128), dis, b2)
    return out.reshape(NPAD)[:N_NODES, None]


# speedup vs baseline: 153.5502x; 1.1256x over previous
import functools

import jax
import jax.numpy as jnp
from jax import lax
from jax.experimental import pallas as pl
from jax.experimental.pallas import tpu as pltpu
from jax.experimental.pallas import tpu_sc as plsc

N_NODES = 100000
N_EDGES = 1600000

NCORES = 2
NSUB = 16
NWORK = NCORES * NSUB

NPAD = 100352
R = NPAD // 128
STRIPE = NPAD // NSUB

CH = 2000
OUTER = 25
EDGES_PER_WORKER = CH * OUTER
assert NWORK * EDGES_PER_WORKER == N_EDGES


def _sc_mesh():
    return plsc.VectorSubcoreMesh(core_axis_name="c", subcore_axis_name="s")


def _sc_agg(edge_flat, ys, zeros1, ones_h, with_deg):
    nf = len(ys)
    nout = nf + (1 if with_deg else 0)
    out_t = [jax.ShapeDtypeStruct((NCORES * NPAD,), jnp.float32)] * nout
    scratch = (
        [pltpu.VMEM((CH,), jnp.int32)] * 2
        + [pltpu.VMEM((CH,), jnp.float32) for _ in range(nf)]
        + [pltpu.VMEM((CH,), jnp.float32)]
        + [pltpu.VMEM((STRIPE,), jnp.float32)]
        + [pltpu.VMEM_SHARED((NPAD,), jnp.float32) for _ in range(nf + nout)]
    )

    @functools.partial(
        pl.kernel,
        mesh=_sc_mesh(),
        out_type=out_t,
        scratch_types=scratch,
    )
    def k(edge_hbm, *rest):
        y_hbm = rest[:nf]
        z_hbm = rest[nf]
        ones_hbm = rest[nf + 1]
        outs = rest[nf + 2:nf + 2 + nout]
        rowv, colv = rest[nf + 2 + nout:nf + 4 + nout]
        gbufs = rest[nf + 4 + nout:nf + 4 + nout + nf]
        onesv = rest[nf + 4 + nout + nf]
        stripev = rest[nf + 5 + nout + nf]
        ysp = rest[nf + 6 + nout + nf:nf + 6 + nout + 2 * nf]
        accs = rest[nf + 6 + nout + 2 * nf:]

        cid = lax.axis_index("c")
        sid = lax.axis_index("s")
        t0 = sid * STRIPE
        sl = pl.ds(t0, STRIPE)
        pltpu.sync_copy(z_hbm.at[sl], stripev)
        for a in accs:
            pltpu.sync_copy(stripev, a.at[sl])
        for yh, yp in zip(y_hbm, ysp):
            pltpu.sync_copy(yh.at[sl], stripev)
            pltpu.sync_copy(stripev, yp.at[sl])
        if with_deg:
            pltpu.sync_copy(ones_hbm, onesv)
        plsc.subcore_barrier()

        wid = cid * NSUB + sid

        def body(b, carry):
            e0 = (b * NWORK + wid) * CH
            if nf:
                pltpu.sync_copy(edge_hbm.at[pl.ds(e0, CH)], rowv)
            pltpu.sync_copy(edge_hbm.at[pl.ds(N_EDGES + e0, CH)], colv)
            for f in range(nf):
                pltpu.sync_copy(ysp[f].at[rowv], gbufs[f])
                pltpu.sync_copy(gbufs[f], accs[f].at[colv], add=True)
            if with_deg:
                pltpu.sync_copy(onesv, accs[nf].at[colv], add=True)
            return carry

        lax.fori_loop(0, OUTER, body, 0)
        plsc.subcore_barrier()
        for a, o in zip(accs, outs):
            pltpu.sync_copy(a.at[sl], stripev)
            pltpu.sync_copy(stripev, o.at[pl.ds(cid * NPAD + t0, STRIPE)])

    outs = k(edge_flat, *ys, zeros1, ones_h)
    return outs if isinstance(outs, (list, tuple)) else [outs]


def _tc1(degp, x0, x1):
    def body(degp_ref, x0_ref, x1_ref, dis_ref, y0_ref, y1_ref):
        deg = degp_ref[0] + degp_ref[1]
        dis = jnp.where(deg > 0, lax.rsqrt(deg), 0.0)
        dis_ref[...] = dis
        y0_ref[...] = dis * x0_ref[...]
        y1_ref[...] = dis * x1_ref[...]

    s = jax.ShapeDtypeStruct((R, 128), jnp.float32)
    return pl.pallas_call(body, out_shape=[s, s, s])(degp, x0, x1)


def _tc2(s10, s11, dis, W1, b1, W2):
    def body(s10_ref, s11_ref, dis_ref, w1_ref, b1_ref, w2_ref, y2_ref):
        dis = dis_ref[...]
        a0 = (s10_ref[0] + s10_ref[1]) * dis
        a1 = (s11_ref[0] + s11_ref[1]) * dis
        w1 = w1_ref[...]
        b1 = b1_ref[...]
        w2 = w2_ref[...]
        v = jnp.zeros_like(a0)
        for k in range(64):
            h = jnp.maximum(a0 * w1[0, k] + a1 * w1[1, k] + b1[k], 0.0)
            v = v + h * w2[k, 0]
        y2_ref[...] = dis * v

    RB = R // 7

    return pl.pallas_call(
        body,
        grid=(7,),
        in_specs=[
            pl.BlockSpec((NCORES, RB, 128), lambda i: (0, i, 0)),
            pl.BlockSpec((NCORES, RB, 128), lambda i: (0, i, 0)),
            pl.BlockSpec((RB, 128), lambda i: (i, 0)),
            pl.BlockSpec((2, 64), lambda i: (0, 0)),
            pl.BlockSpec((64,), lambda i: (0,)),
            pl.BlockSpec((64, 1), lambda i: (0, 0)),
        ],
        out_specs=pl.BlockSpec((RB, 128), lambda i: (i, 0)),
        out_shape=jax.ShapeDtypeStruct((R, 128), jnp.float32),
    )(s10, s11, dis, W1, b1, W2)


def _tc3(s2p, dis, b2):
    def body(s2p_ref, dis_ref, b2_ref, out_ref):
        s2 = s2p_ref[0] + s2p_ref[1]
        out_ref[...] = jnp.maximum(dis_ref[...] * s2 + b2_ref[0], 0.0)

    return pl.pallas_call(
        body, out_shape=jax.ShapeDtypeStruct((R, 128), jnp.float32),
    )(s2p, dis, b2)


def kernel(x, edge_index, W1, b1, W2, b2):
    edge_flat = edge_index.reshape(2 * N_EDGES)
    xpad = jnp.zeros((NPAD, 2), jnp.float32).at[:N_NODES].set(x)
    x0 = xpad[:, 0].reshape(R, 128)
    x1 = xpad[:, 1].reshape(R, 128)
    z1 = jnp.zeros((NPAD,), jnp.float32)
    ones_h = jnp.ones((CH,), jnp.float32)

    (degp,) = _sc_agg(edge_flat, [], z1, ones_h, True)
    dis, y0, y1 = _tc1(degp.reshape(NCORES, R, 128), x0, x1)
    s10, s11 = _sc_agg(edge_flat, [y0.reshape(NPAD), y1.reshape(NPAD)],
                       z1, ones_h, False)
    y2 = _tc2(s10.reshape(NCORES, R, 128), s11.reshape(NCORES, R, 128),
              dis, W1, b1, W2)
    (s2p,) = _sc_agg(edge_flat, [y2.reshape(NPAD)], z1, ones_h, False)
    out = _tc3(s2p.reshape(NCORES, R, 128), dis, b2)
    return out.reshape(NPAD)[:N_NODES, None]
